# baseline (speedup 1.0000x reference)
import functools

import jax
import jax.numpy as jnp
from jax import lax
from jax.experimental import pallas as pl
from jax.experimental.pallas import tpu as pltpu
from jax.experimental.pallas import tpu_sc as plsc

N = 10000
E = 160000
D = 256
H = 256
CLS = 10
G = 64

NC = 2
NS = 16
LANES = 16

CHUNK = 128
E_PAD = 163840
IDX_ROWS = E_PAD // CHUNK
AGG_ROWS = IDX_ROWS // NS
DEG_ROWS = IDX_ROWS // (NC * NS)
NBUF = 2
NIDX = 4
HHALF = H // 2
ACC_ROWS = N + 16
ROWS_PER_TILE = 632
DEG_BINS = 10240
DEG_PER_TILE = DEG_BINS // NS
ROWB = 1000


_SC_MESH = plsc.VectorSubcoreMesh(core_axis_name="c", subcore_axis_name="s")


@functools.partial(
    pl.kernel,
    out_type=jax.ShapeDtypeStruct((NC, DEG_BINS), jnp.float32),
    mesh=_SC_MESH,
    scratch_types=[
        pltpu.VMEM((DEG_ROWS, CHUNK), jnp.int32),
        pltpu.VMEM((CHUNK,), jnp.float32),
        pltpu.VMEM((DEG_PER_TILE,), jnp.float32),
        pltpu.VMEM_SHARED((DEG_BINS,), jnp.float32),
    ],
)
def _deg_kernel(dstp_hbm, out_hbm, dst_v, ones_v, zeros_v, deg_sh):
    cid = lax.axis_index("c")
    sid = lax.axis_index("s")
    wid = sid * NC + cid
    for i in range(CHUNK // LANES):
        ones_v[pl.ds(i * LANES, LANES)] = jnp.ones((LANES,), jnp.float32)
    for i in range(DEG_PER_TILE // LANES):
        zeros_v[pl.ds(i * LANES, LANES)] = jnp.zeros((LANES,), jnp.float32)
    pltpu.sync_copy(zeros_v, deg_sh.at[pl.ds(sid * DEG_PER_TILE, DEG_PER_TILE)])
    pltpu.sync_copy(dstp_hbm.at[pl.ds(wid * DEG_ROWS, DEG_ROWS)], dst_v)
    plsc.subcore_barrier()

    def body(j, carry):
        pltpu.sync_copy(ones_v, deg_sh.at[dst_v.at[j]], add=True)
        return carry

    lax.fori_loop(0, DEG_ROWS, body, 0)
    plsc.subcore_barrier()
    pltpu.sync_copy(
        deg_sh.at[pl.ds(sid * DEG_PER_TILE, DEG_PER_TILE)],
        out_hbm.at[cid].at[pl.ds(sid * DEG_PER_TILE, DEG_PER_TILE)],
    )


@functools.partial(
    pl.kernel,
    out_type=jax.ShapeDtypeStruct((NC, N, HHALF), jnp.float32),
    mesh=_SC_MESH,
    scratch_types=[
        pltpu.VMEM((NIDX, 2, CHUNK), jnp.int32),
        pltpu.VMEM((NBUF, CHUNK, HHALF), jnp.float32),
        pltpu.VMEM_SHARED((ACC_ROWS, HHALF), jnp.float32),
        [pltpu.SemaphoreType.DMA] * NIDX,
        [pltpu.SemaphoreType.DMA] * NBUF,
    ],
)
def _agg_kernel(hs_hbm, idxp_hbm, out_hbm,
                iring, gbuf, acc_sh, isems, gsems):
    cid = lax.axis_index("c")
    sid = lax.axis_index("s")
    base = sid * AGG_ROWS
    hs_c = hs_hbm.at[cid]
    rb = pl.multiple_of(
        jnp.minimum(sid * ROWS_PER_TILE, N - ROWS_PER_TILE), 8)
    pltpu.sync_copy(hs_c.at[pl.ds(rb, ROWS_PER_TILE)],
                    acc_sh.at[pl.ds(rb, ROWS_PER_TILE)])
    for t in range(NIDX - 1):
        pltpu.async_copy(idxp_hbm.at[base + t], iring.at[t], isems[t])
    plsc.subcore_barrier()
    pltpu.make_async_copy(idxp_hbm.at[base], iring.at[0], isems[0]).wait()
    pltpu.async_copy(hs_c.at[iring.at[0].at[0]], gbuf.at[0], gsems[0])

    def body(i, carry):
        for t in range(NIDX):
            jj = i * NIDX + t
            b = t % NBUF
            bn = (t + 1) % NBUF
            tp = (t + 3) % NIDX
            tn = (t + 1) % NIDX
            pltpu.make_async_copy(hs_c.at[iring.at[t].at[0]],
                                  gbuf.at[b], gsems[b]).wait()
            pltpu.sync_copy(gbuf.at[b], acc_sh.at[iring.at[t].at[1]],
                            add=True)

            @pl.when(jj + 3 < AGG_ROWS)
            def _():
                pltpu.async_copy(idxp_hbm.at[base + jj + 3],
                                 iring.at[tp], isems[tp])

            @pl.when(jj + 1 < AGG_ROWS)
            def _():
                pltpu.make_async_copy(idxp_hbm.at[base], iring.at[tn],
                                      isems[tn]).wait()
                pltpu.async_copy(hs_c.at[iring.at[tn].at[0]],
                                 gbuf.at[bn], gsems[bn])

        return carry

    lax.fori_loop(0, AGG_ROWS // NIDX, body, 0)
    plsc.subcore_barrier()
    pltpu.sync_copy(acc_sh.at[pl.ds(rb, ROWS_PER_TILE)],
                    out_hbm.at[cid].at[pl.ds(rb, ROWS_PER_TILE)])


def _split2(out_ref, h):
    out_ref[0] = h[:, :HHALF]
    out_ref[1] = h[:, HHALF:]


def _cat2(acc_ref):
    return jnp.concatenate([acc_ref[0], acc_ref[1]], axis=1)


def _mm_first_body(x_ref, dis_ref, w_ref, out_ref):
    h = jnp.dot(x_ref[...], w_ref[...], preferred_element_type=jnp.float32)
    _split2(out_ref, h * dis_ref[...])


def _mm_first(x, dis, W):
    return pl.pallas_call(
        _mm_first_body,
        grid=(N // ROWB,),
        in_specs=[
            pl.BlockSpec((ROWB, D), lambda i: (i, 0)),
            pl.BlockSpec((ROWB, 1), lambda i: (i, 0)),
            pl.BlockSpec((D, H), lambda i: (0, 0)),
        ],
        out_specs=pl.BlockSpec((NC, ROWB, HHALF), lambda i: (0, i, 0)),
        out_shape=jax.ShapeDtypeStruct((NC, N, HHALF), jnp.float32),
    )(x, dis, W)


def _mm_mid_body(acc_ref, dis_ref, b_ref, w_ref, out_ref):
    pre = jnp.maximum(dis_ref[...] * _cat2(acc_ref) + b_ref[...], 0.0)
    h = jnp.dot(pre, w_ref[...], preferred_element_type=jnp.float32)
    _split2(out_ref, h * dis_ref[...])


def _mm_mid(acc, dis, b, W):
    return pl.pallas_call(
        _mm_mid_body,
        grid=(N // ROWB,),
        in_specs=[
            pl.BlockSpec((NC, ROWB, HHALF), lambda i: (0, i, 0)),
            pl.BlockSpec((ROWB, 1), lambda i: (i, 0)),
            pl.BlockSpec((1, H), lambda i: (0, 0)),
            pl.BlockSpec((H, H), lambda i: (0, 0)),
        ],
        out_specs=pl.BlockSpec((NC, ROWB, HHALF), lambda i: (0, i, 0)),
        out_shape=jax.ShapeDtypeStruct((NC, N, HHALF), jnp.float32),
    )(acc, dis, b, W)


def _final_body(acc_ref, dis_ref, b_ref, batch_ref, wl_ref, bl_ref,
                out_ref, acc_scr):
    i = pl.program_id(0)
    x3 = dis_ref[...] * _cat2(acc_ref) + b_ref[...]
    gids = batch_ref[0]
    onehot = (lax.broadcasted_iota(jnp.int32, (G, ROWB), 0) == gids)
    onehot = onehot.astype(jnp.float32)
    z = jnp.concatenate([x3, jnp.ones((ROWB, 128), jnp.float32)], axis=1)
    part = jnp.dot(onehot, z, preferred_element_type=jnp.float32)

    @pl.when(i == 0)
    def _():
        acc_scr[...] = jnp.zeros_like(acc_scr)

    acc_scr[...] += part

    @pl.when(i == pl.num_programs(0) - 1)
    def _():
        sums = acc_scr[:, :H]
        cnt = acc_scr[:, H:H + 1]
        pooled = sums / jnp.maximum(cnt, 1.0)
        out_ref[...] = (
            jnp.dot(pooled, wl_ref[...], preferred_element_type=jnp.float32)
            + bl_ref[...])


def _final(acc, dis, b, batch3, Wl, bl):
    return pl.pallas_call(
        _final_body,
        grid=(N // ROWB,),
        in_specs=[
            pl.BlockSpec((NC, ROWB, HHALF), lambda i: (0, i, 0)),
            pl.BlockSpec((ROWB, 1), lambda i: (i, 0)),
            pl.BlockSpec((1, H), lambda i: (0, 0)),
            pl.BlockSpec((1, 1, ROWB), lambda i: (i, 0, 0)),
            pl.BlockSpec((H, CLS), lambda i: (0, 0)),
            pl.BlockSpec((1, CLS), lambda i: (0, 0)),
        ],
        out_specs=pl.BlockSpec((G, CLS), lambda i: (0, 0)),
        out_shape=jax.ShapeDtypeStruct((G, CLS), jnp.float32),
        scratch_shapes=[pltpu.VMEM((G, H + 128), jnp.float32)],
    )(acc, dis, b, batch3, Wl, bl)


def kernel(x, edge_index, batch, W1, b1, W2, b2, W3, b3, Wl, bl):
    pad = E_PAD - E
    srcp = jnp.concatenate(
        [edge_index[0], jnp.zeros((pad,), jnp.int32)]).reshape(IDX_ROWS, CHUNK)
    dstp = jnp.concatenate(
        [edge_index[1], jnp.full((pad,), N, jnp.int32)]).reshape(IDX_ROWS, CHUNK)
    idxp = jnp.stack([srcp, dstp], axis=1)

    degp = _deg_kernel(dstp)
    deg = degp[0, :N] + degp[1, :N] + 1.0
    dis = lax.rsqrt(deg).reshape(N, 1)

    batch3 = batch.reshape(N // ROWB, 1, ROWB)

    hs = _mm_first(x, dis, W1)
    a1 = _agg_kernel(hs, idxp)
    hs = _mm_mid(a1, dis, b1.reshape(1, H), W2)
    a2 = _agg_kernel(hs, idxp)
    hs = _mm_mid(a2, dis, b2.reshape(1, H), W3)
    a3 = _agg_kernel(hs, idxp)
    return _final(a3, dis, b3.reshape(1, H), batch3, Wl, bl.reshape(1, CLS))


# speedup vs baseline: 1.2043x; 1.2043x over previous
import functools

import jax
import jax.numpy as jnp
from jax import lax
from jax.experimental import pallas as pl
from jax.experimental.pallas import tpu as pltpu
from jax.experimental.pallas import tpu_sc as plsc

N = 10000
E = 160000
D = 256
H = 256
CLS = 10
G = 64

NC = 2
NS = 16
LANES = 16

CHUNK = 128
E_PAD = 163840
IDX_ROWS = E_PAD // CHUNK
AGG_ROWS = IDX_ROWS // NS
DEG_ROWS = IDX_ROWS // (NC * NS)
NBUF = 2
NIDX = 4
HHALF = H // 2
ACC_ROWS = N + 16
ROWS_PER_TILE = 632
DEG_BINS = 10240
DEG_PER_TILE = DEG_BINS // NS
ROWB = 1000


_SC_MESH = plsc.VectorSubcoreMesh(core_axis_name="c", subcore_axis_name="s")


@functools.partial(
    pl.kernel,
    out_type=jax.ShapeDtypeStruct((NC, DEG_BINS), jnp.float32),
    mesh=_SC_MESH,
    scratch_types=[
        pltpu.VMEM((DEG_ROWS, CHUNK), jnp.int32),
        pltpu.VMEM((CHUNK,), jnp.float32),
        pltpu.VMEM((DEG_PER_TILE,), jnp.float32),
        pltpu.VMEM_SHARED((DEG_BINS,), jnp.float32),
    ],
)
def _deg_kernel(dstp_hbm, out_hbm, dst_v, ones_v, zeros_v, deg_sh):
    cid = lax.axis_index("c")
    sid = lax.axis_index("s")
    wid = sid * NC + cid
    for i in range(CHUNK // LANES):
        ones_v[pl.ds(i * LANES, LANES)] = jnp.ones((LANES,), jnp.float32)
    for i in range(DEG_PER_TILE // LANES):
        zeros_v[pl.ds(i * LANES, LANES)] = jnp.zeros((LANES,), jnp.float32)
    pltpu.sync_copy(zeros_v, deg_sh.at[pl.ds(sid * DEG_PER_TILE, DEG_PER_TILE)])
    pltpu.sync_copy(dstp_hbm.at[pl.ds(wid * DEG_ROWS, DEG_ROWS)], dst_v)
    plsc.subcore_barrier()

    def body(j, carry):
        pltpu.sync_copy(ones_v, deg_sh.at[dst_v.at[j]], add=True)
        return carry

    lax.fori_loop(0, DEG_ROWS, body, 0)
    plsc.subcore_barrier()
    pltpu.sync_copy(
        deg_sh.at[pl.ds(sid * DEG_PER_TILE, DEG_PER_TILE)],
        out_hbm.at[cid].at[pl.ds(sid * DEG_PER_TILE, DEG_PER_TILE)],
    )


@functools.partial(
    pl.kernel,
    out_type=jax.ShapeDtypeStruct((NC, N, HHALF), jnp.float32),
    mesh=_SC_MESH,
    scratch_types=[
        pltpu.VMEM((NIDX, 2, CHUNK), jnp.int32),
        pltpu.VMEM((NBUF, CHUNK, HHALF), jnp.float32),
        pltpu.VMEM_SHARED((ACC_ROWS, HHALF), jnp.float32),
        [pltpu.SemaphoreType.DMA] * NIDX,
        [pltpu.SemaphoreType.DMA] * NBUF,
    ],
)
def _agg_kernel(hs_hbm, idxp_hbm, out_hbm,
                iring, gbuf, acc_sh, isems, gsems):
    cid = lax.axis_index("c")
    sid = lax.axis_index("s")
    base = sid * AGG_ROWS
    hs_c = hs_hbm.at[cid]
    rb = pl.multiple_of(
        jnp.minimum(sid * ROWS_PER_TILE, N - ROWS_PER_TILE), 8)
    pltpu.sync_copy(hs_c.at[pl.ds(rb, ROWS_PER_TILE)],
                    acc_sh.at[pl.ds(rb, ROWS_PER_TILE)])
    for t in range(NIDX):
        pltpu.async_copy(idxp_hbm.at[base + t], iring.at[t], isems[t])
    plsc.subcore_barrier()
    for b in range(NBUF):
        pltpu.make_async_copy(idxp_hbm.at[base], iring.at[b], isems[b]).wait()
        pltpu.async_copy(hs_c.at[iring.at[b].at[0]], gbuf.at[b], gsems[b])

    def body(i, carry):
        for t in range(NIDX):
            jj = i * NIDX + t
            b = t % NBUF
            tn = (t + 2) % NIDX
            pltpu.make_async_copy(hs_c.at[iring.at[t].at[0]],
                                  gbuf.at[b], gsems[b]).wait()
            pltpu.sync_copy(gbuf.at[b], acc_sh.at[iring.at[t].at[1]],
                            add=True)

            @pl.when(jj + NIDX < AGG_ROWS)
            def _():
                pltpu.async_copy(idxp_hbm.at[base + jj + NIDX],
                                 iring.at[t], isems[t])

            @pl.when(jj + 2 < AGG_ROWS)
            def _():
                pltpu.make_async_copy(idxp_hbm.at[base], iring.at[tn],
                                      isems[tn]).wait()
                pltpu.async_copy(hs_c.at[iring.at[tn].at[0]],
                                 gbuf.at[b], gsems[b])

        return carry

    lax.fori_loop(0, AGG_ROWS // NIDX, body, 0)
    plsc.subcore_barrier()
    pltpu.sync_copy(acc_sh.at[pl.ds(rb, ROWS_PER_TILE)],
                    out_hbm.at[cid].at[pl.ds(rb, ROWS_PER_TILE)])


def _split2(out_ref, h):
    out_ref[0] = h[:, :HHALF]
    out_ref[1] = h[:, HHALF:]


def _cat2(acc_ref):
    return jnp.concatenate([acc_ref[0], acc_ref[1]], axis=1)


def _mm_first_body(x_ref, dis_ref, w_ref, out_ref):
    h = jnp.dot(x_ref[...], w_ref[...], preferred_element_type=jnp.float32)
    _split2(out_ref, h * dis_ref[...])


def _mm_first(x, dis, W):
    return pl.pallas_call(
        _mm_first_body,
        grid=(N // ROWB,),
        in_specs=[
            pl.BlockSpec((ROWB, D), lambda i: (i, 0)),
            pl.BlockSpec((ROWB, 1), lambda i: (i, 0)),
            pl.BlockSpec((D, H), lambda i: (0, 0)),
        ],
        out_specs=pl.BlockSpec((NC, ROWB, HHALF), lambda i: (0, i, 0)),
        out_shape=jax.ShapeDtypeStruct((NC, N, HHALF), jnp.float32),
    )(x, dis, W)


def _mm_mid_body(acc_ref, dis_ref, b_ref, w_ref, out_ref):
    pre = jnp.maximum(dis_ref[...] * _cat2(acc_ref) + b_ref[...], 0.0)
    h = jnp.dot(pre, w_ref[...], preferred_element_type=jnp.float32)
    _split2(out_ref, h * dis_ref[...])


def _mm_mid(acc, dis, b, W):
    return pl.pallas_call(
        _mm_mid_body,
        grid=(N // ROWB,),
        in_specs=[
            pl.BlockSpec((NC, ROWB, HHALF), lambda i: (0, i, 0)),
            pl.BlockSpec((ROWB, 1), lambda i: (i, 0)),
            pl.BlockSpec((1, H), lambda i: (0, 0)),
            pl.BlockSpec((H, H), lambda i: (0, 0)),
        ],
        out_specs=pl.BlockSpec((NC, ROWB, HHALF), lambda i: (0, i, 0)),
        out_shape=jax.ShapeDtypeStruct((NC, N, HHALF), jnp.float32),
    )(acc, dis, b, W)


def _final_body(acc_ref, dis_ref, b_ref, batch_ref, wl_ref, bl_ref,
                out_ref, acc_scr):
    i = pl.program_id(0)
    x3 = dis_ref[...] * _cat2(acc_ref) + b_ref[...]
    gids = batch_ref[0]
    onehot = (lax.broadcasted_iota(jnp.int32, (G, ROWB), 0) == gids)
    onehot = onehot.astype(jnp.float32)
    z = jnp.concatenate([x3, jnp.ones((ROWB, 128), jnp.float32)], axis=1)
    part = jnp.dot(onehot, z, preferred_element_type=jnp.float32)

    @pl.when(i == 0)
    def _():
        acc_scr[...] = jnp.zeros_like(acc_scr)

    acc_scr[...] += part

    @pl.when(i == pl.num_programs(0) - 1)
    def _():
        sums = acc_scr[:, :H]
        cnt = acc_scr[:, H:H + 1]
        pooled = sums / jnp.maximum(cnt, 1.0)
        out_ref[...] = (
            jnp.dot(pooled, wl_ref[...], preferred_element_type=jnp.float32)
            + bl_ref[...])


def _final(acc, dis, b, batch3, Wl, bl):
    return pl.pallas_call(
        _final_body,
        grid=(N // ROWB,),
        in_specs=[
            pl.BlockSpec((NC, ROWB, HHALF), lambda i: (0, i, 0)),
            pl.BlockSpec((ROWB, 1), lambda i: (i, 0)),
            pl.BlockSpec((1, H), lambda i: (0, 0)),
            pl.BlockSpec((1, 1, ROWB), lambda i: (i, 0, 0)),
            pl.BlockSpec((H, CLS), lambda i: (0, 0)),
            pl.BlockSpec((1, CLS), lambda i: (0, 0)),
        ],
        out_specs=pl.BlockSpec((G, CLS), lambda i: (0, 0)),
        out_shape=jax.ShapeDtypeStruct((G, CLS), jnp.float32),
        scratch_shapes=[pltpu.VMEM((G, H + 128), jnp.float32)],
    )(acc, dis, b, batch3, Wl, bl)


def kernel(x, edge_index, batch, W1, b1, W2, b2, W3, b3, Wl, bl):
    pad = E_PAD - E
    srcp = jnp.concatenate(
        [edge_index[0], jnp.zeros((pad,), jnp.int32)]).reshape(IDX_ROWS, CHUNK)
    dstp = jnp.concatenate(
        [edge_index[1], jnp.full((pad,), N, jnp.int32)]).reshape(IDX_ROWS, CHUNK)
    idxp = jnp.stack([srcp, dstp], axis=1)

    degp = _deg_kernel(dstp)
    deg = degp[0, :N] + degp[1, :N] + 1.0
    dis = lax.rsqrt(deg).reshape(N, 1)

    batch3 = batch.reshape(N // ROWB, 1, ROWB)

    hs = _mm_first(x, dis, W1)
    a1 = _agg_kernel(hs, idxp)
    hs = _mm_mid(a1, dis, b1.reshape(1, H), W2)
    a2 = _agg_kernel(hs, idxp)
    hs = _mm_mid(a2, dis, b2.reshape(1, H), W3)
    a3 = _agg_kernel(hs, idxp)
    return _final(a3, dis, b3.reshape(1, H), batch3, Wl, bl.reshape(1, CLS))


# speedup vs baseline: 1.2142x; 1.0082x over previous
import functools

import jax
import jax.numpy as jnp
from jax import lax
from jax.experimental import pallas as pl
from jax.experimental.pallas import tpu as pltpu
from jax.experimental.pallas import tpu_sc as plsc

N = 10000
E = 160000
D = 256
H = 256
CLS = 10
G = 64

NC = 2
NS = 16
LANES = 16

CHUNK = 128
E_PAD = 163840
IDX_ROWS = E_PAD // CHUNK
AGG_ROWS = IDX_ROWS // NS
DEG_ROWS = IDX_ROWS // (NC * NS)
NBUF = 3
NIDX = 6
HHALF = H // 2
ACC_ROWS = N + 16
ROWS_PER_TILE = 632
DEG_BINS = 10240
DEG_PER_TILE = DEG_BINS // NS
ROWB = 1000


_SC_MESH = plsc.VectorSubcoreMesh(core_axis_name="c", subcore_axis_name="s")


@functools.partial(
    pl.kernel,
    out_type=jax.ShapeDtypeStruct((NC, DEG_BINS), jnp.float32),
    mesh=_SC_MESH,
    scratch_types=[
        pltpu.VMEM((DEG_ROWS, CHUNK), jnp.int32),
        pltpu.VMEM((CHUNK,), jnp.float32),
        pltpu.VMEM((DEG_PER_TILE,), jnp.float32),
        pltpu.VMEM_SHARED((DEG_BINS,), jnp.float32),
    ],
)
def _deg_kernel(dstp_hbm, out_hbm, dst_v, ones_v, zeros_v, deg_sh):
    cid = lax.axis_index("c")
    sid = lax.axis_index("s")
    wid = sid * NC + cid
    for i in range(CHUNK // LANES):
        ones_v[pl.ds(i * LANES, LANES)] = jnp.ones((LANES,), jnp.float32)
    for i in range(DEG_PER_TILE // LANES):
        zeros_v[pl.ds(i * LANES, LANES)] = jnp.zeros((LANES,), jnp.float32)
    pltpu.sync_copy(zeros_v, deg_sh.at[pl.ds(sid * DEG_PER_TILE, DEG_PER_TILE)])
    pltpu.sync_copy(dstp_hbm.at[pl.ds(wid * DEG_ROWS, DEG_ROWS)], dst_v)
    plsc.subcore_barrier()

    def body(j, carry):
        pltpu.sync_copy(ones_v, deg_sh.at[dst_v.at[j]], add=True)
        return carry

    lax.fori_loop(0, DEG_ROWS, body, 0)
    plsc.subcore_barrier()
    pltpu.sync_copy(
        deg_sh.at[pl.ds(sid * DEG_PER_TILE, DEG_PER_TILE)],
        out_hbm.at[cid].at[pl.ds(sid * DEG_PER_TILE, DEG_PER_TILE)],
    )


@functools.partial(
    pl.kernel,
    out_type=jax.ShapeDtypeStruct((NC, N, HHALF), jnp.float32),
    mesh=_SC_MESH,
    scratch_types=[
        pltpu.VMEM((NIDX, 2, CHUNK), jnp.int32),
        pltpu.VMEM((NBUF, CHUNK, HHALF), jnp.float32),
        pltpu.VMEM_SHARED((ACC_ROWS, HHALF), jnp.float32),
        [pltpu.SemaphoreType.DMA] * NIDX,
        [pltpu.SemaphoreType.DMA] * NBUF,
    ],
)
def _agg_kernel(hs_hbm, idxp_hbm, out_hbm,
                iring, gbuf, acc_sh, isems, gsems):
    cid = lax.axis_index("c")
    sid = lax.axis_index("s")
    base = sid * AGG_ROWS
    hs_c = hs_hbm.at[cid]
    rb = pl.multiple_of(
        jnp.minimum(sid * ROWS_PER_TILE, N - ROWS_PER_TILE), 8)
    pltpu.sync_copy(hs_c.at[pl.ds(rb, ROWS_PER_TILE)],
                    acc_sh.at[pl.ds(rb, ROWS_PER_TILE)])
    for t in range(NIDX):
        pltpu.async_copy(idxp_hbm.at[base + t], iring.at[t], isems[t])
    plsc.subcore_barrier()
    for b in range(NBUF):
        pltpu.make_async_copy(idxp_hbm.at[base], iring.at[b], isems[b]).wait()
        pltpu.async_copy(hs_c.at[iring.at[b].at[0]], gbuf.at[b], gsems[b])

    def step(jj, t6):
        b = t6 % NBUF
        tn = (t6 + 3) % NIDX
        pltpu.make_async_copy(hs_c.at[iring.at[t6].at[0]],
                              gbuf.at[b], gsems[b]).wait()
        pltpu.sync_copy(gbuf.at[b], acc_sh.at[iring.at[t6].at[1]],
                        add=True)

        @pl.when(jj + NIDX < AGG_ROWS)
        def _():
            pltpu.async_copy(idxp_hbm.at[base + jj + NIDX],
                             iring.at[t6], isems[t6])

        @pl.when(jj + NBUF < AGG_ROWS)
        def _():
            pltpu.make_async_copy(idxp_hbm.at[base], iring.at[tn],
                                  isems[tn]).wait()
            pltpu.async_copy(hs_c.at[iring.at[tn].at[0]],
                             gbuf.at[b], gsems[b])

    def body(i, carry):
        for t6 in range(NIDX):
            step(i * NIDX + t6, t6)
        return carry

    lax.fori_loop(0, (AGG_ROWS - 2) // NIDX, body, 0)
    for t6 in range(2):
        step(AGG_ROWS - 2 + t6, t6)
    plsc.subcore_barrier()
    pltpu.sync_copy(acc_sh.at[pl.ds(rb, ROWS_PER_TILE)],
                    out_hbm.at[cid].at[pl.ds(rb, ROWS_PER_TILE)])


def _split2(out_ref, h):
    out_ref[0] = h[:, :HHALF]
    out_ref[1] = h[:, HHALF:]


def _cat2(acc_ref):
    return jnp.concatenate([acc_ref[0], acc_ref[1]], axis=1)


def _mm_first_body(x_ref, dis_ref, w_ref, out_ref):
    h = jnp.dot(x_ref[...], w_ref[...], preferred_element_type=jnp.float32)
    _split2(out_ref, h * dis_ref[...])


def _mm_first(x, dis, W):
    return pl.pallas_call(
        _mm_first_body,
        grid=(N // ROWB,),
        in_specs=[
            pl.BlockSpec((ROWB, D), lambda i: (i, 0)),
            pl.BlockSpec((ROWB, 1), lambda i: (i, 0)),
            pl.BlockSpec((D, H), lambda i: (0, 0)),
        ],
        out_specs=pl.BlockSpec((NC, ROWB, HHALF), lambda i: (0, i, 0)),
        out_shape=jax.ShapeDtypeStruct((NC, N, HHALF), jnp.float32),
    )(x, dis, W)


def _mm_mid_body(acc_ref, dis_ref, b_ref, w_ref, out_ref):
    pre = jnp.maximum(dis_ref[...] * _cat2(acc_ref) + b_ref[...], 0.0)
    h = jnp.dot(pre, w_ref[...], preferred_element_type=jnp.float32)
    _split2(out_ref, h * dis_ref[...])


def _mm_mid(acc, dis, b, W):
    return pl.pallas_call(
        _mm_mid_body,
        grid=(N // ROWB,),
        in_specs=[
            pl.BlockSpec((NC, ROWB, HHALF), lambda i: (0, i, 0)),
            pl.BlockSpec((ROWB, 1), lambda i: (i, 0)),
            pl.BlockSpec((1, H), lambda i: (0, 0)),
            pl.BlockSpec((H, H), lambda i: (0, 0)),
        ],
        out_specs=pl.BlockSpec((NC, ROWB, HHALF), lambda i: (0, i, 0)),
        out_shape=jax.ShapeDtypeStruct((NC, N, HHALF), jnp.float32),
    )(acc, dis, b, W)


def _final_body(acc_ref, dis_ref, b_ref, batch_ref, wl_ref, bl_ref,
                out_ref, acc_scr):
    i = pl.program_id(0)
    x3 = dis_ref[...] * _cat2(acc_ref) + b_ref[...]
    gids = batch_ref[0]
    onehot = (lax.broadcasted_iota(jnp.int32, (G, ROWB), 0) == gids)
    onehot = onehot.astype(jnp.float32)
    z = jnp.concatenate([x3, jnp.ones((ROWB, 128), jnp.float32)], axis=1)
    part = jnp.dot(onehot, z, preferred_element_type=jnp.float32)

    @pl.when(i == 0)
    def _():
        acc_scr[...] = jnp.zeros_like(acc_scr)

    acc_scr[...] += part

    @pl.when(i == pl.num_programs(0) - 1)
    def _():
        sums = acc_scr[:, :H]
        cnt = acc_scr[:, H:H + 1]
        pooled = sums / jnp.maximum(cnt, 1.0)
        out_ref[...] = (
            jnp.dot(pooled, wl_ref[...], preferred_element_type=jnp.float32)
            + bl_ref[...])


def _final(acc, dis, b, batch3, Wl, bl):
    return pl.pallas_call(
        _final_body,
        grid=(N // ROWB,),
        in_specs=[
            pl.BlockSpec((NC, ROWB, HHALF), lambda i: (0, i, 0)),
            pl.BlockSpec((ROWB, 1), lambda i: (i, 0)),
            pl.BlockSpec((1, H), lambda i: (0, 0)),
            pl.BlockSpec((1, 1, ROWB), lambda i: (i, 0, 0)),
            pl.BlockSpec((H, CLS), lambda i: (0, 0)),
            pl.BlockSpec((1, CLS), lambda i: (0, 0)),
        ],
        out_specs=pl.BlockSpec((G, CLS), lambda i: (0, 0)),
        out_shape=jax.ShapeDtypeStruct((G, CLS), jnp.float32),
        scratch_shapes=[pltpu.VMEM((G, H + 128), jnp.float32)],
    )(acc, dis, b, batch3, Wl, bl)


def kernel(x, edge_index, batch, W1, b1, W2, b2, W3, b3, Wl, bl):
    pad = E_PAD - E
    srcp = jnp.concatenate(
        [edge_index[0], jnp.zeros((pad,), jnp.int32)]).reshape(IDX_ROWS, CHUNK)
    dstp = jnp.concatenate(
        [edge_index[1], jnp.full((pad,), N, jnp.int32)]).reshape(IDX_ROWS, CHUNK)
    idxp = jnp.stack([srcp, dstp], axis=1)

    degp = _deg_kernel(dstp)
    deg = degp[0, :N] + degp[1, :N] + 1.0
    dis = lax.rsqrt(deg).reshape(N, 1)

    batch3 = batch.reshape(N // ROWB, 1, ROWB)

    hs = _mm_first(x, dis, W1)
    a1 = _agg_kernel(hs, idxp)
    hs = _mm_mid(a1, dis, b1.reshape(1, H), W2)
    a2 = _agg_kernel(hs, idxp)
    hs = _mm_mid(a2, dis, b2.reshape(1, H), W3)
    a3 = _agg_kernel(hs, idxp)
    return _final(a3, dis, b3.reshape(1, H), batch3, Wl, bl.reshape(1, CLS))
